# baseline (speedup 1.0000x reference)
import functools

import jax
import jax.numpy as jnp
from jax import lax
from jax.experimental import pallas as pl
from jax.experimental.pallas import tpu as pltpu
from jax.experimental.pallas import tpu_sc as plsc

_N = 16384
_D = 128
_K = 20
_KP1 = _K + 1

_NC = 2
_NS = 16
_L = 16
_NW = _NC * _NS
_EX_PER_TILE = _N // _NW
_G = 16
_GROUPS = _EX_PER_TILE // _G
_ROWS = _G * _KP1
_IDX_PER_TILE = _EX_PER_TILE * _KP1
_HT_BLK = _D * _G
_CHUNK = 16
_NCHUNK = _ROWS // _CHUNK


def _sc_scores_kernel(table_hbm, idx_hbm, ht_hbm, pos_hbm, neg_hbm,
                      idx_v, rows_v0, rows_v1, ht_v0, ht_v1, pos_v, neg_v,
                      gsems, hsems):
    rows_bufs = (rows_v0, rows_v1)
    ht_bufs = (ht_v0, ht_v1)
    wid = lax.axis_index("s") * _NC + lax.axis_index("c")
    ex_base = wid * _EX_PER_TILE
    blk_base = wid * _GROUPS

    pltpu.sync_copy(idx_hbm.at[pl.ds(wid * _IDX_PER_TILE, _IDX_PER_TILE)],
                    idx_v)

    iota = lax.iota(jnp.int32, _L)
    row_base = iota * _KP1

    def _issue(g, b):
        goff = g * _ROWS

        @pl.loop(0, _ROWS // _L)
        def _rowgrp(sg):
            vec = idx_v[pl.ds(goff + sg * _L, _L)]
            for j in range(_L):
                pltpu.async_copy(table_hbm.at[vec[j]],
                                 rows_bufs[b].at[sg * _L + j], gsems[b])

        pltpu.async_copy(ht_hbm.at[pl.ds((blk_base + g) * _HT_BLK, _HT_BLK)],
                         ht_bufs[b], hsems[b])

    def _drain(g, b):
        pltpu.make_async_copy(table_hbm.at[pl.ds(0, _ROWS)], rows_bufs[b],
                              gsems[b]).wait()
        pltpu.make_async_copy(
            ht_hbm.at[pl.ds((blk_base + g) * _HT_BLK, _HT_BLK)],
            ht_bufs[b], hsems[b]).wait()

    _issue(0, 0)
    _issue(1, 1)

    @pl.loop(0, _GROUPS, step=2)
    def _groups(g0):
        for b in range(2):
            g = g0 + b
            _drain(g, b)

            def _dot_step(d, accs):
                hcol = ht_bufs[b][pl.ds(d * _L, _L)]
                dvec = jnp.full((_L,), d, jnp.int32)
                new = []
                for k in range(_KP1):
                    col = plsc.load_gather(rows_bufs[b], [row_base + k, dvec])
                    new.append(accs[k] + col * hcol)
                return tuple(new)

            accs = lax.fori_loop(
                0, _D, _dot_step,
                tuple(jnp.zeros((_L,), jnp.float32) for _ in range(_KP1)))

            @pl.when(g + 2 < _GROUPS)
            def _():
                _issue(g + 2, b)

            pos_v[...] = accs[0]
            for k in range(1, _KP1):
                neg_v[pl.ds((k - 1) * _L, _L)] = accs[k]
            pltpu.sync_copy(pos_v, pos_hbm.at[pl.ds(ex_base + g * _G, _G)])
            pltpu.sync_copy(neg_v,
                            neg_hbm.at[pl.ds((blk_base + g) * _G * _K,
                                             _G * _K)])


@functools.partial(
    pl.kernel,
    out_type=(
        jax.ShapeDtypeStruct((_N,), jnp.float32),
        jax.ShapeDtypeStruct((_N * _K,), jnp.float32),
    ),
    mesh=plsc.VectorSubcoreMesh(core_axis_name="c", subcore_axis_name="s",
                                num_cores=_NC, num_subcores=_NS),
    compiler_params=pltpu.CompilerParams(needs_layout_passes=False),
    scratch_types=[
        pltpu.VMEM((_IDX_PER_TILE,), jnp.int32),
        pltpu.VMEM((_ROWS, _D), jnp.float32),
        pltpu.VMEM((_ROWS, _D), jnp.float32),
        pltpu.VMEM((_HT_BLK,), jnp.float32),
        pltpu.VMEM((_HT_BLK,), jnp.float32),
        pltpu.VMEM((_G,), jnp.float32),
        pltpu.VMEM((_G * _K,), jnp.float32),
        (pltpu.SemaphoreType.DMA, pltpu.SemaphoreType.DMA),
        (pltpu.SemaphoreType.DMA, pltpu.SemaphoreType.DMA),
    ],
)
def _sc_scores(table_hbm, idx_hbm, ht_hbm, pos_hbm, neg_hbm,
               idx_v, rows_v0, rows_v1, ht_v0, ht_v1, pos_v, neg_v,
               gsems, hsems):
    _sc_scores_kernel(table_hbm, idx_hbm, ht_hbm, pos_hbm, neg_hbm,
                      idx_v, rows_v0, rows_v1, ht_v0, ht_v1, pos_v, neg_v,
                      gsems, hsems)


def _tc_loss_body(pos_ref, neg_ref, out_ref):
    pos = pos_ref[...]
    neg = neg_ref[...]
    total = jnp.sum(jax.nn.log_sigmoid(pos))
    total = total + jnp.sum(jax.nn.log_sigmoid(-neg)) * (1.0 / _K)
    out_ref[0, 0] = -total


def kernel(hidden_state, label_idxex, neg_idxes, out_word_embeddings):
    idx_all = jnp.concatenate(
        [label_idxex.astype(jnp.int32)[:, None],
         neg_idxes.astype(jnp.int32)],
        axis=1).reshape(_N * _KP1)
    ht_flat = (hidden_state.reshape(_N // _G, _G, _D)
               .transpose(0, 2, 1).reshape(_N * _D))

    pos, neg = _sc_scores(out_word_embeddings, idx_all, ht_flat)

    loss = pl.pallas_call(
        _tc_loss_body,
        out_shape=jax.ShapeDtypeStruct((1, 1), jnp.float32),
        out_specs=pl.BlockSpec(memory_space=pltpu.SMEM),
    )(pos.reshape(_N // _D, _D), neg.reshape(_N * _K // _D, _D))
    return loss[0, 0]


# speedup vs baseline: 1.0211x; 1.0211x over previous
import functools

import jax
import jax.numpy as jnp
from jax import lax
from jax.experimental import pallas as pl
from jax.experimental.pallas import tpu as pltpu
from jax.experimental.pallas import tpu_sc as plsc

_N = 16384
_D = 128
_K = 20
_KP1 = _K + 1

_NC = 2
_NS = 16
_L = 16
_NW = _NC * _NS
_EX_PER_TILE = _N // _NW
_G = 16
_GROUPS = _EX_PER_TILE // _G
_ROWS = _G * _KP1
_IDX_PER_TILE = _EX_PER_TILE * _KP1
_HT_BLK = _D * _G
_CHUNK = 16
_NCHUNK = _ROWS // _CHUNK


def _sc_scores_kernel(table_hbm, idx_hbm, ht_hbm, pos_hbm, neg_hbm,
                      idx_v, rows_v0, rows_v1, ht_v0, ht_v1, pos_v, neg_v,
                      gsems, hsems, dsems):
    rows_bufs = (rows_v0, rows_v1)
    ht_bufs = (ht_v0, ht_v1)
    wid = lax.axis_index("s") * _NC + lax.axis_index("c")
    ex_base = wid * _EX_PER_TILE
    blk_base = wid * _GROUPS

    pltpu.sync_copy(idx_hbm.at[pl.ds(wid * _IDX_PER_TILE, _IDX_PER_TILE)],
                    idx_v)

    iota = lax.iota(jnp.int32, _L)
    row_base = iota * _KP1

    _NSTREAM = 160

    def _issue(g, b):
        goff = g * _ROWS
        for j in range(_NSTREAM // _CHUNK):
            pltpu.async_copy(
                table_hbm.at[idx_v.at[pl.ds(goff + j * _CHUNK, _CHUNK)]],
                rows_bufs[b].at[pl.ds(j * _CHUNK, _CHUNK)], gsems[b])

        @pl.loop(0, (_ROWS - _NSTREAM) // _L)
        def _rowgrp(sg):
            vec = idx_v[pl.ds(goff + _NSTREAM + sg * _L, _L)]
            for j in range(_L):
                pltpu.async_copy(table_hbm.at[vec[j]],
                                 rows_bufs[b].at[_NSTREAM + sg * _L + j],
                                 dsems[b])

        pltpu.async_copy(ht_hbm.at[pl.ds((blk_base + g) * _HT_BLK, _HT_BLK)],
                         ht_bufs[b], hsems[b])

    def _drain(g, b):
        goff = g * _ROWS
        for j in range(_NSTREAM // _CHUNK):
            pltpu.make_async_copy(
                table_hbm.at[idx_v.at[pl.ds(goff + j * _CHUNK, _CHUNK)]],
                rows_bufs[b].at[pl.ds(j * _CHUNK, _CHUNK)], gsems[b]).wait()
        pltpu.make_async_copy(
            table_hbm.at[pl.ds(0, _ROWS - _NSTREAM)],
            rows_bufs[b].at[pl.ds(_NSTREAM, _ROWS - _NSTREAM)],
            dsems[b]).wait()
        pltpu.make_async_copy(
            ht_hbm.at[pl.ds((blk_base + g) * _HT_BLK, _HT_BLK)],
            ht_bufs[b], hsems[b]).wait()

    _issue(0, 0)
    _issue(1, 1)

    @pl.loop(0, _GROUPS, step=2)
    def _groups(g0):
        for b in range(2):
            g = g0 + b
            _drain(g, b)

            def _dot_step(d, accs):
                hcol = ht_bufs[b][pl.ds(d * _L, _L)]
                dvec = jnp.full((_L,), d, jnp.int32)
                new = []
                for k in range(_KP1):
                    col = plsc.load_gather(rows_bufs[b], [row_base + k, dvec])
                    new.append(accs[k] + col * hcol)
                return tuple(new)

            accs = lax.fori_loop(
                0, _D, _dot_step,
                tuple(jnp.zeros((_L,), jnp.float32) for _ in range(_KP1)))

            @pl.when(g + 2 < _GROUPS)
            def _():
                _issue(g + 2, b)

            pos_v[...] = accs[0]
            for k in range(1, _KP1):
                neg_v[pl.ds((k - 1) * _L, _L)] = accs[k]
            pltpu.sync_copy(pos_v, pos_hbm.at[pl.ds(ex_base + g * _G, _G)])
            pltpu.sync_copy(neg_v,
                            neg_hbm.at[pl.ds((blk_base + g) * _G * _K,
                                             _G * _K)])


@functools.partial(
    pl.kernel,
    out_type=(
        jax.ShapeDtypeStruct((_N,), jnp.float32),
        jax.ShapeDtypeStruct((_N * _K,), jnp.float32),
    ),
    mesh=plsc.VectorSubcoreMesh(core_axis_name="c", subcore_axis_name="s",
                                num_cores=_NC, num_subcores=_NS),
    compiler_params=pltpu.CompilerParams(needs_layout_passes=False),
    scratch_types=[
        pltpu.VMEM((_IDX_PER_TILE,), jnp.int32),
        pltpu.VMEM((_ROWS, _D), jnp.float32),
        pltpu.VMEM((_ROWS, _D), jnp.float32),
        pltpu.VMEM((_HT_BLK,), jnp.float32),
        pltpu.VMEM((_HT_BLK,), jnp.float32),
        pltpu.VMEM((_G,), jnp.float32),
        pltpu.VMEM((_G * _K,), jnp.float32),
        (pltpu.SemaphoreType.DMA, pltpu.SemaphoreType.DMA),
        (pltpu.SemaphoreType.DMA, pltpu.SemaphoreType.DMA),
        (pltpu.SemaphoreType.DMA, pltpu.SemaphoreType.DMA),
    ],
)
def _sc_scores(table_hbm, idx_hbm, ht_hbm, pos_hbm, neg_hbm,
               idx_v, rows_v0, rows_v1, ht_v0, ht_v1, pos_v, neg_v,
               gsems, hsems, dsems):
    _sc_scores_kernel(table_hbm, idx_hbm, ht_hbm, pos_hbm, neg_hbm,
                      idx_v, rows_v0, rows_v1, ht_v0, ht_v1, pos_v, neg_v,
                      gsems, hsems, dsems)


def _tc_loss_body(pos_ref, neg_ref, out_ref):
    pos = pos_ref[...]
    neg = neg_ref[...]
    total = jnp.sum(jax.nn.log_sigmoid(pos))
    total = total + jnp.sum(jax.nn.log_sigmoid(-neg)) * (1.0 / _K)
    out_ref[0, 0] = -total


def kernel(hidden_state, label_idxex, neg_idxes, out_word_embeddings):
    idx_all = jnp.concatenate(
        [label_idxex.astype(jnp.int32)[:, None],
         neg_idxes.astype(jnp.int32)],
        axis=1).reshape(_N * _KP1)
    ht_flat = (hidden_state.reshape(_N // _G, _G, _D)
               .transpose(0, 2, 1).reshape(_N * _D))

    pos, neg = _sc_scores(out_word_embeddings, idx_all, ht_flat)

    loss = pl.pallas_call(
        _tc_loss_body,
        out_shape=jax.ShapeDtypeStruct((1, 1), jnp.float32),
        out_specs=pl.BlockSpec(memory_space=pltpu.SMEM),
    )(pos.reshape(_N // _D, _D), neg.reshape(_N * _K // _D, _D))
    return loss[0, 0]
